# baseline (speedup 1.0000x reference)
import functools

import jax
import jax.numpy as jnp
from jax import lax
from jax.experimental import pallas as pl
from jax.experimental.pallas import tpu as pltpu
from jax.experimental.pallas import tpu_sc as plsc

B = 16384
DX = 512
DY = 64
RPS = 8192
NT = 4
NC = 2
NS = 16
NW = NC * NS
BPW = B // NW
KC = 64
LCAP = BPW + KC
WMAX = 12

_MESH = plsc.VectorSubcoreMesh(core_axis_name="c", subcore_axis_name="s")


@functools.partial(
    pl.kernel,
    mesh=_MESH,
    compiler_params=pltpu.CompilerParams(use_tc_tiling_on_sc=False,
                                         needs_layout_passes=False),
    out_type=jax.ShapeDtypeStruct((NW, 2 * WMAX + 1, 1, KC), jnp.int32),
    scratch_types=[
        pltpu.VMEM((BPW,), jnp.int32),
        pltpu.VMEM((NT, LCAP), jnp.int32),
        pltpu.VMEM((NT, LCAP), jnp.int32),
        pltpu.VMEM((2 * WMAX + 1, 1, KC), jnp.int32),
    ],
)
def _partition(idx_hbm, sched_hbm, idx_v, loc2, pos1, sched_v):
    wid = lax.axis_index("s") * NC + lax.axis_index("c")
    base = wid * BPW
    pltpu.sync_copy(idx_hbm.at[pl.ds(base, BPW)], idx_v)

    lanes = lax.iota(jnp.int32, 16)
    zero = jnp.int32(0)

    def part_body(i, offs):
        iv = idx_v[pl.ds(i * 16, 16)]
        tid = lax.shift_right_logical(iv, 13)
        loc = lax.bitwise_and(iv, RPS - 1)
        pos = base + i * 16 + lanes
        new = []
        for t in range(NT):
            m = tid == t
            cum = plsc.cumsum(m.astype(jnp.int32))
            dst = offs[t] + cum - 1
            plsc.store_scatter(loc2.at[t], [dst], loc, mask=m)
            plsc.store_scatter(pos1.at[t], [dst], pos, mask=m)
            new.append(offs[t] + cum[15])
        return tuple(new)

    offs = lax.fori_loop(0, BPW // 16, part_body, (zero, zero, zero, zero))

    ones = jnp.full((16,), 1, jnp.int32)
    nw = zero
    for t in range(NT):
        c = offs[t]

        fill = ones * jnp.maximum(c - 1, 0)
        lastl = plsc.load_gather(loc2.at[t], [fill])
        lastp = plsc.load_gather(pos1.at[t], [fill])
        for r in range(KC // 16):
            padidx = c + r * 16 + lanes
            plsc.store_scatter(loc2.at[t], [padidx], lastl)
            plsc.store_scatter(pos1.at[t], [padidx], lastp)

        s = jnp.maximum(lax.bitwise_and(c + 7, ~jnp.int32(7)), jnp.int32(KC))
        nwin_t = (c + KC - 1) // KC
        last0 = s - KC

        def wbody(w, nw_c, t=t, last0=last0):
            start = pl.multiple_of(jnp.minimum(w * KC, last0), 8)
            for r in range(KC // 16):
                vl = loc2[t, pl.ds(start + r * 16, 16)]
                vp = pos1[t, pl.ds(start + r * 16, 16)]
                plsc.store_scatter(sched_v.at[nw_c, 0], [r * 16 + lanes], vl)
                plsc.store_scatter(sched_v.at[WMAX + nw_c, 0],
                                   [r * 16 + lanes], vp)
            plsc.store_scatter(sched_v.at[2 * WMAX, 0], [ones + nw_c],
                               jnp.full((16,), t, jnp.int32))
            return nw_c + 1

        nw = lax.fori_loop(0, nwin_t, wbody, nw)

    plsc.store_scatter(sched_v.at[2 * WMAX, 0], [ones * 0], ones * nw)
    pltpu.sync_copy(sched_v, sched_hbm.at[wid])


def _make_gather(d, use_tc_tiling, stacked=False):
    @functools.partial(
        pl.kernel,
        mesh=_MESH,
        compiler_params=pltpu.CompilerParams(
            use_tc_tiling_on_sc=use_tc_tiling, needs_layout_passes=False),
        out_type=jax.ShapeDtypeStruct((B, d), jnp.float32),
        scratch_types=[
            pltpu.VMEM((2 * WMAX + 1, 1, KC), jnp.int32),
            pltpu.VMEM((KC, d), jnp.float32),
            pltpu.VMEM((KC, d), jnp.float32),
            pltpu.SemaphoreType.DMA,
            pltpu.SemaphoreType.DMA,
            pltpu.SemaphoreType.DMA,
        ],
    )
    def _gather(*args):
        if stacked:
            (tstk, sched_hbm, out, sched, st0, st1,
             semg, sems0, sems1) = args
            tabs = tuple(tstk.at[t] for t in range(NT))
        else:
            (t0, t1, t2, t3, sched_hbm, out, sched, st0, st1,
             semg, sems0, sems1) = args
            tabs = (t0, t1, t2, t3)
        wid = lax.axis_index("s") * NC + lax.axis_index("c")
        pltpu.sync_copy(sched_hbm.at[wid], sched)

        nwin = sched[2 * WMAX, 0, pl.ds(0, 16)][0]
        onesv = jnp.full((16,), 1, jnp.int32)

        def run_window(w, stp, semsp):
            ids = sched.at[w, 0]
            prow = sched.at[WMAX + w, 0]
            tw = plsc.load_gather(sched.at[2 * WMAX, 0], [onesv + w])[0]
            for t in range(NT):
                @pl.when(tw == t)
                def _(t=t):
                    @pl.when(w >= 2)
                    def _drain():
                        pltpu.make_async_copy(stp, out.at[prow], semsp).wait()
                    pltpu.async_copy(tabs[t].at[ids], stp, semg).wait()
                    pltpu.async_copy(stp, out.at[prow], semsp)

        def gs_body(k, carry):
            run_window(2 * k, st0, sems0)

            @pl.when(2 * k + 1 < nwin)
            def _odd():
                run_window(2 * k + 1, st1, sems1)
            return carry

        lax.fori_loop(0, (nwin + 1) // 2, gs_body, 0)

        prow0 = sched.at[WMAX, 0]
        pltpu.make_async_copy(st0, out.at[prow0], sems0).wait()
        pltpu.make_async_copy(st1, out.at[prow0], sems1).wait()

    return _gather


_gather_x = _make_gather(DX, True)
_gather_y = _make_gather(DY, False, stacked=True)


def kernel(x0, x1, x2, x3, y0, y1, y2, y3, random_idx):
    idx = random_idx.astype(jnp.int32)
    sched = _partition(idx)
    bx = _gather_x(x0, x1, x2, x3, sched)
    ys = jnp.stack([y0, y1, y2, y3])
    by = _gather_y(ys, sched)
    return (bx, by)


# speedup vs baseline: 1.2368x; 1.2368x over previous
import functools

import jax
import jax.numpy as jnp
from jax import lax
from jax.experimental import pallas as pl
from jax.experimental.pallas import tpu as pltpu
from jax.experimental.pallas import tpu_sc as plsc

B = 16384
DX = 512
DY = 64
RPS = 8192
NT = 4
NC = 2
NS = 16
NW = NC * NS
BPW = B // NW
KC = 64
LCAP = BPW + KC
WMAX = 12

_MESH = plsc.VectorSubcoreMesh(core_axis_name="c", subcore_axis_name="s")


@functools.partial(
    pl.kernel,
    mesh=_MESH,
    compiler_params=pltpu.CompilerParams(use_tc_tiling_on_sc=False,
                                         needs_layout_passes=False),
    out_type=jax.ShapeDtypeStruct((NW, 2 * WMAX + 1, 1, KC), jnp.int32),
    scratch_types=[
        pltpu.VMEM((BPW,), jnp.int32),
        pltpu.VMEM((NT, LCAP), jnp.int32),
        pltpu.VMEM((NT, LCAP), jnp.int32),
        pltpu.VMEM((2 * WMAX + 1, 1, KC), jnp.int32),
    ],
)
def _partition(idx_hbm, sched_hbm, idx_v, loc2, pos1, sched_v):
    wid = lax.axis_index("s") * NC + lax.axis_index("c")
    base = wid * BPW
    pltpu.sync_copy(idx_hbm.at[pl.ds(base, BPW)], idx_v)

    lanes = lax.iota(jnp.int32, 16)
    zero = jnp.int32(0)

    def part_body(i, offs):
        iv = idx_v[pl.ds(i * 16, 16)]
        tid = lax.shift_right_logical(iv, 13)
        loc = lax.bitwise_and(iv, RPS - 1)
        pos = base + i * 16 + lanes
        new = []
        for t in range(NT):
            m = tid == t
            cum = plsc.cumsum(m.astype(jnp.int32))
            dst = offs[t] + cum - 1
            plsc.store_scatter(loc2.at[t], [dst], loc, mask=m)
            plsc.store_scatter(pos1.at[t], [dst], pos, mask=m)
            new.append(offs[t] + cum[15])
        return tuple(new)

    offs = lax.fori_loop(0, BPW // 16, part_body, (zero, zero, zero, zero))

    ones = jnp.full((16,), 1, jnp.int32)
    nw = zero
    for t in range(NT):
        c = offs[t]

        fill = ones * jnp.maximum(c - 1, 0)
        lastl = plsc.load_gather(loc2.at[t], [fill])
        lastp = plsc.load_gather(pos1.at[t], [fill])
        for r in range(KC // 16):
            padidx = c + r * 16 + lanes
            plsc.store_scatter(loc2.at[t], [padidx], lastl)
            plsc.store_scatter(pos1.at[t], [padidx], lastp)

        s = jnp.maximum(lax.bitwise_and(c + 7, ~jnp.int32(7)), jnp.int32(KC))
        nwin_t = (c + KC - 1) // KC
        last0 = s - KC

        def wbody(w, nw_c, t=t, last0=last0):
            start = pl.multiple_of(jnp.minimum(w * KC, last0), 8)
            for r in range(KC // 16):
                vl = loc2[t, pl.ds(start + r * 16, 16)]
                vp = pos1[t, pl.ds(start + r * 16, 16)]
                plsc.store_scatter(sched_v.at[nw_c, 0], [r * 16 + lanes], vl)
                plsc.store_scatter(sched_v.at[WMAX + nw_c, 0],
                                   [r * 16 + lanes], vp)
            plsc.store_scatter(sched_v.at[2 * WMAX, 0], [ones + nw_c],
                               jnp.full((16,), t, jnp.int32))
            return nw_c + 1

        nw = lax.fori_loop(0, nwin_t, wbody, nw)

    plsc.store_scatter(sched_v.at[2 * WMAX, 0], [ones * 0], ones * nw)
    pltpu.sync_copy(sched_v, sched_hbm.at[wid])


def _make_gather(d, use_tc_tiling):
    @functools.partial(
        pl.kernel,
        mesh=_MESH,
        compiler_params=pltpu.CompilerParams(
            use_tc_tiling_on_sc=use_tc_tiling, needs_layout_passes=False),
        out_type=jax.ShapeDtypeStruct((B, d), jnp.float32),
        scratch_types=[
            pltpu.VMEM((2 * WMAX + 1, 1, KC), jnp.int32),
            pltpu.VMEM((KC, d), jnp.float32),
            pltpu.VMEM((KC, d), jnp.float32),
            pltpu.SemaphoreType.DMA,
            pltpu.SemaphoreType.DMA,
            pltpu.SemaphoreType.DMA,
        ],
    )
    def _gather(t0, t1, t2, t3, sched_hbm, out,
                sched, st0, st1, semg, sems0, sems1):
        tabs = (t0, t1, t2, t3)
        wid = lax.axis_index("s") * NC + lax.axis_index("c")
        pltpu.sync_copy(sched_hbm.at[wid], sched)

        nwin = sched[2 * WMAX, 0, pl.ds(0, 16)][0]
        onesv = jnp.full((16,), 1, jnp.int32)

        def run_window(w, stp, semsp):
            ids = sched.at[w, 0]
            prow = sched.at[WMAX + w, 0]
            tw = plsc.load_gather(sched.at[2 * WMAX, 0], [onesv + w])[0]
            for t in range(NT):
                @pl.when(tw == t)
                def _(t=t):
                    @pl.when(w >= 2)
                    def _drain():
                        pltpu.make_async_copy(stp, out.at[prow], semsp).wait()
                    pltpu.async_copy(tabs[t].at[ids], stp, semg).wait()
                    pltpu.async_copy(stp, out.at[prow], semsp)

        def gs_body(k, carry):
            run_window(2 * k, st0, sems0)

            @pl.when(2 * k + 1 < nwin)
            def _odd():
                run_window(2 * k + 1, st1, sems1)
            return carry

        lax.fori_loop(0, (nwin + 1) // 2, gs_body, 0)

        prow0 = sched.at[WMAX, 0]
        pltpu.make_async_copy(st0, out.at[prow0], sems0).wait()
        pltpu.make_async_copy(st1, out.at[prow0], sems1).wait()

    return _gather


_gather_x = _make_gather(DX, True)
_gather_y = _make_gather(DY, False)


def kernel(x0, x1, x2, x3, y0, y1, y2, y3, random_idx):
    idx = random_idx.astype(jnp.int32)
    sched = _partition(idx)
    bx = _gather_x(x0, x1, x2, x3, sched)
    ys = jnp.stack([y0, y1, y2, y3])
    by = _gather_y(ys[0], ys[1], ys[2], ys[3], sched)
    return (bx, by)
